# Initial kernel scaffold; baseline (speedup 1.0000x reference)
import jax
import jax.numpy as jnp
from jax.experimental import pallas as pl


def kernel(positions, edge_index, shifts, ptr, voigts):
    raise NotImplementedError("write your pallas kernel here")


# speedup vs baseline: 3.9142x; 3.9142x over previous
import functools

import jax
import jax.numpy as jnp
import numpy as np
from jax import lax
from jax.experimental import pallas as pl
from jax.experimental.pallas import tpu as pltpu
from jax.experimental.pallas import tpu_sc as plsc

N_NODES_K = 100000
N_EDGES_K = 6400000
N_GRAPHS_K = 128

NUM_CORES = 2
NUM_SUBCORES = 16
NUM_TILES = NUM_CORES * NUM_SUBCORES
EDGES_PER_TILE = N_EDGES_K // NUM_TILES
CHUNK = 2000
NCHUNKS = EDGES_PER_TILE // CHUNK
INNER = CHUNK // 16

_PERM = np.array([0, 5, 4, 5, 1, 3, 4, 3, 2], dtype=np.int32)
_STRESS_IDX = (6 * np.repeat(np.arange(N_GRAPHS_K, dtype=np.int32), 9)
               + np.tile(_PERM, N_GRAPHS_K)).astype(np.int32)


def _rsqrt_len(l2):
    bits = plsc.bitcast(l2, jnp.int32)
    y = plsc.bitcast(jnp.full((16,), 0x5F3759DF, jnp.int32)
                     - lax.shift_right_logical(bits, 1), jnp.float32)
    xhalf = l2 * 0.5
    y = y * (1.5 - xhalf * y * y)
    y = y * (1.5 - xhalf * y * y)
    y = y * (1.5 - xhalf * y * y)
    ln = l2 * y
    return jnp.where(l2 > 0.0, ln, 0.0)


def _body(pos_hbm, send_hbm, recv_hbm, shifts_hbm, ptr_hbm, voigt_hbm,
          sidx_hbm, vec_out, len_out, nat_out, stress_out,
          sidx_v, ridx_v, srows_v, rrows_v, shv_v, vecv_v, lenv_v,
          ptr_v, nat_v, voigt_v, sperm_v, stress_v, sem):
    wid = lax.axis_index("s") * NUM_CORES + lax.axis_index("c")
    iota = lax.iota(jnp.int32, 16)

    c0 = jnp.full((16,), 0, jnp.int32)
    c1 = jnp.full((16,), 1, jnp.int32)
    c2 = jnp.full((16,), 2, jnp.int32)

    def chunk_body(j, _):
        base = wid * EDGES_PER_TILE + j * CHUNK
        pltpu.sync_copy(send_hbm.at[pl.ds(base, CHUNK)], sidx_v)
        pltpu.sync_copy(recv_hbm.at[pl.ds(base, CHUNK)], ridx_v)
        g1 = pltpu.async_copy(pos_hbm.at[sidx_v], srows_v, sem)
        g2 = pltpu.async_copy(pos_hbm.at[ridx_v], rrows_v, sem)
        pltpu.sync_copy(shifts_hbm.at[pl.ds(base * 3, CHUNK * 3)], shv_v)
        g1.wait()
        g2.wait()

        def edge_body(t, _):
            ii = t * 16 + iota
            i3 = ii * 3
            vx = (plsc.load_gather(rrows_v, [ii, c0])
                  - plsc.load_gather(srows_v, [ii, c0])
                  + plsc.load_gather(shv_v, [i3]))
            vy = (plsc.load_gather(rrows_v, [ii, c1])
                  - plsc.load_gather(srows_v, [ii, c1])
                  + plsc.load_gather(shv_v, [i3 + 1]))
            vz = (plsc.load_gather(rrows_v, [ii, c2])
                  - plsc.load_gather(srows_v, [ii, c2])
                  + plsc.load_gather(shv_v, [i3 + 2]))
            plsc.store_scatter(vecv_v, [i3], vx)
            plsc.store_scatter(vecv_v, [i3 + 1], vy)
            plsc.store_scatter(vecv_v, [i3 + 2], vz)
            l2 = vx * vx + vy * vy + vz * vz
            lenv_v[pl.ds(t * 16, 16)] = _rsqrt_len(l2)
            return ()

        lax.fori_loop(0, INNER, edge_body, (), unroll=4)
        pltpu.sync_copy(vecv_v, vec_out.at[pl.ds(base * 3, CHUNK * 3)])
        pltpu.sync_copy(lenv_v, len_out.at[pl.ds(base, CHUNK)])
        return ()

    lax.fori_loop(0, NCHUNKS, chunk_body, ())

    @pl.when(wid == 0)
    def _tiny():
        pltpu.sync_copy(ptr_hbm, ptr_v)

        def nat_body(i, _):
            a = plsc.load_gather(ptr_v, [i * 16 + iota])
            b = plsc.load_gather(ptr_v, [i * 16 + 1 + iota])
            nat_v[pl.ds(i * 16, 16)] = b - a
            return ()

        lax.fori_loop(0, N_GRAPHS_K // 16, nat_body, ())
        pltpu.sync_copy(nat_v, nat_out)

        pltpu.sync_copy(voigt_hbm, voigt_v)
        pltpu.sync_copy(sidx_hbm, sperm_v)

        def stress_body(k, _):
            iv = sperm_v[pl.ds(k * 16, 16)]
            stress_v[pl.ds(k * 16, 16)] = plsc.load_gather(voigt_v, [iv])
            return ()

        lax.fori_loop(0, (N_GRAPHS_K * 9) // 16, stress_body, ())
        pltpu.sync_copy(stress_v, stress_out)


@jax.jit
def _run(positions, sender, receiver, shifts, ptr, voigts_flat, stress_idx):
    mesh = plsc.VectorSubcoreMesh(core_axis_name="c", subcore_axis_name="s",
                                  num_cores=NUM_CORES,
                                  num_subcores=NUM_SUBCORES)
    f = pl.kernel(
        _body,
        out_type=[
            jax.ShapeDtypeStruct((N_EDGES_K * 3,), jnp.float32),
            jax.ShapeDtypeStruct((N_EDGES_K,), jnp.float32),
            jax.ShapeDtypeStruct((N_GRAPHS_K,), jnp.int32),
            jax.ShapeDtypeStruct((N_GRAPHS_K * 9,), jnp.float32),
        ],
        mesh=mesh,
        scratch_types=[
            pltpu.VMEM((CHUNK,), jnp.int32),
            pltpu.VMEM((CHUNK,), jnp.int32),
            pltpu.VMEM((CHUNK, 8), jnp.float32),
            pltpu.VMEM((CHUNK, 8), jnp.float32),
            pltpu.VMEM((CHUNK * 3,), jnp.float32),
            pltpu.VMEM((CHUNK * 3,), jnp.float32),
            pltpu.VMEM((CHUNK,), jnp.float32),
            pltpu.VMEM((N_GRAPHS_K + 1,), jnp.int32),
            pltpu.VMEM((N_GRAPHS_K,), jnp.int32),
            pltpu.VMEM((N_GRAPHS_K * 6,), jnp.float32),
            pltpu.VMEM((N_GRAPHS_K * 9,), jnp.int32),
            pltpu.VMEM((N_GRAPHS_K * 9,), jnp.float32),
            pltpu.SemaphoreType.DMA,
        ],
        compiler_params=pltpu.CompilerParams(needs_layout_passes=False, use_tc_tiling_on_sc=False),
    )
    return f(positions, sender, receiver, shifts, ptr, voigts_flat,
             stress_idx)


def kernel(positions, edge_index, shifts, ptr, voigts):
    sender = edge_index[0].astype(jnp.int32)
    receiver = edge_index[1].astype(jnp.int32)
    ptr32 = ptr.astype(jnp.int32)
    voigts_flat = voigts.reshape(-1)
    stress_idx = jnp.asarray(_STRESS_IDX)
    pos8 = jnp.pad(positions, ((0, 0), (0, 5)))
    vec, lengths, num_atoms, stress = _run(
        pos8, sender, receiver, shifts.reshape(-1), ptr32, voigts_flat,
        stress_idx)
    return (vec.reshape(N_EDGES_K, 3), lengths.reshape(-1, 1), num_atoms,
            stress.reshape(N_GRAPHS_K, 3, 3))
